# baseline (speedup 1.0000x reference)
import functools

import jax
import jax.numpy as jnp
from jax import lax
from jax.experimental import pallas as pl
from jax.experimental.pallas import tpu as pltpu
from jax.experimental.pallas import tpu_sc as plsc

EMB = 16
NE = 8
NC, NS = 2, 16
NW = NC * NS


def _moe_body(n_tok, x_hbm, wrx_hbm, we_hbm, be_hbm, out_hbm,
              x_v, wrx_v, we_v, be_v, out_v):
    tok_w = n_tok // NW
    words = tok_w * EMB
    blocks = tok_w // 16

    cid = lax.axis_index("c")
    sid = lax.axis_index("s")
    wid = sid * NC + cid

    pltpu.sync_copy(x_hbm.at[pl.ds(wid * words, words)], x_v)
    pltpu.sync_copy(wrx_hbm, wrx_v)
    pltpu.sync_copy(we_hbm, we_v)
    pltpu.sync_copy(be_hbm, be_v)

    def bq(v):
        u = plsc.bitcast(v, jnp.int32)
        r = u + jnp.int32(0x7FFF) + ((u >> 16) & 1)
        return plsc.bitcast(r & jnp.int32(-65536), jnp.float32)

    def vexp(v):
        t = v * jnp.float32(1.4426950408889634)
        t = jnp.maximum(t, jnp.float32(-126.0))
        magic = jnp.float32(12582912.0)
        n = (t + magic) - magic
        f = t - n
        ni = n.astype(jnp.int32)
        c1 = jnp.float32(0.6931471805599453)
        c2 = jnp.float32(0.2402265069591007)
        c3 = jnp.float32(0.05550410866482158)
        c4 = jnp.float32(0.009618129107628477)
        c5 = jnp.float32(0.0013333558146428443)
        c6 = jnp.float32(0.00015403530393381608)
        p = c6
        p = p * f + c5
        p = p * f + c4
        p = p * f + c3
        p = p * f + c2
        p = p * f + c1
        p = p * f + jnp.float32(1.0)
        scale = plsc.bitcast((ni + 127) << 23, jnp.float32)
        return p * scale

    def round_ref(ref, nslices):
        def step(i, carry):
            s = pl.ds(i * 16, 16)
            ref[s] = bq(ref[s])
            return carry
        lax.fori_loop(0, nslices, step, 0)

    round_ref(wrx_v, EMB * NE)
    round_ref(we_v, we_v.shape[0] // 16)

    iota = lax.iota(jnp.int32, 16)
    lane_row = iota * EMB
    neg = jnp.full((16,), -3.0e38, jnp.float32)
    zero = jnp.zeros((16,), jnp.float32)

    def block(b, carry):
        base = lane_row + b * (16 * EMB)
        xT = [bq(plsc.load_gather(x_v, [base + k])) for k in range(EMB)]

        logits = []
        for e in range(NE):
            acc = wrx_v[pl.ds((EMB * NE + e) * 16, 16)]
            for k in range(EMB):
                acc = acc + xT[k] * wrx_v[pl.ds((k * NE + e) * 16, 16)]
            logits.append(acc)

        best1 = logits[0]
        idx1 = jnp.zeros((16,), jnp.int32)
        for e in range(1, NE):
            c = logits[e] > best1
            best1 = jnp.where(c, logits[e], best1)
            idx1 = jnp.where(c, jnp.int32(e), idx1)
        best2 = neg
        idx2 = jnp.zeros((16,), jnp.int32)
        for e in range(NE):
            le = jnp.where(idx1 == e, neg, logits[e])
            c = le > best2
            best2 = jnp.where(c, le, best2)
            idx2 = jnp.where(c, jnp.int32(e), idx2)

        s = zero
        for e in range(NE):
            s = s + vexp(logits[e] - best1)
        r = 1.0 / s
        r = r + r * (jnp.float32(1.0) - s * r)
        g1 = r
        g2 = vexp(best2 - best1) * r

        ebase1 = idx1 * (EMB * EMB)
        ebase2 = idx2 * (EMB * EMB)
        bbase1 = idx1 * EMB
        bbase2 = idx2 * EMB
        for half in range(2):
            hofs = half * (EMB // 2)
            accs = [zero for _ in range(EMB // 2)]
            for k in range(EMB):
                k1 = ebase1 + (k * EMB + hofs)
                k2 = ebase2 + (k * EMB + hofs)
                for d in range(EMB // 2):
                    w1 = plsc.load_gather(we_v, [k1 + d])
                    w2 = plsc.load_gather(we_v, [k2 + d])
                    wc = g1 * w1 + g2 * w2
                    accs[d] = accs[d] + xT[k] * wc
            for d in range(EMB // 2):
                b1 = plsc.load_gather(be_v, [bbase1 + (hofs + d)])
                b2 = plsc.load_gather(be_v, [bbase2 + (hofs + d)])
                o = accs[d] + g1 * b1 + g2 * b2
                plsc.store_scatter(out_v, [base + (hofs + d)], o)
        return carry

    lax.fori_loop(0, blocks, block, 0)
    pltpu.sync_copy(out_v, out_hbm.at[pl.ds(wid * words, words)])


@functools.partial(jax.jit, static_argnames=())
def _moe_call(x_flat, wrx, we_flat, be_flat):
    n_words = x_flat.shape[0]
    n_tok = n_words // EMB
    words = n_words // NW
    body = functools.partial(_moe_body, n_tok)
    fn = pl.kernel(
        body,
        out_type=jax.ShapeDtypeStruct((n_words,), jnp.float32),
        mesh=plsc.VectorSubcoreMesh(core_axis_name="c", subcore_axis_name="s",
                                    num_cores=NC, num_subcores=NS),
        compiler_params=pltpu.CompilerParams(needs_layout_passes=False),
        scratch_types=[
            pltpu.VMEM((words,), jnp.float32),
            pltpu.VMEM((wrx.shape[0],), jnp.float32),
            pltpu.VMEM((we_flat.shape[0],), jnp.float32),
            pltpu.VMEM((be_flat.shape[0],), jnp.float32),
            pltpu.VMEM((words,), jnp.float32),
        ],
    )
    return fn(x_flat, wrx, we_flat, be_flat)


def kernel(x, Wr, br, We, be):
    n, emb = x.shape
    x_flat = x.reshape(-1)
    wr_rows = jnp.concatenate([Wr.reshape(-1), br.reshape(-1)], axis=0)
    wrx = jnp.broadcast_to(wr_rows[:, None], (wr_rows.shape[0], 16)).reshape(-1)
    we_flat = We.reshape(-1)
    be_flat = be.reshape(-1)
    out = _moe_call(x_flat, wrx, we_flat, be_flat)
    return out.reshape(n, emb)


# speedup vs baseline: 2.4258x; 2.4258x over previous
import functools

import jax
import jax.numpy as jnp
import numpy as np
from jax import lax
from jax.experimental import pallas as pl
from jax.experimental.pallas import tpu as pltpu
from jax.experimental.pallas import tpu_sc as plsc

EMB = 16
NE = 8
NC, NS = 2, 16
NW = NC * NS
BN = 4096


def _tc_logits_body(x_ref, wr_ref, br_ref, o_ref):
    o_ref[...] = lax.dot_general(
        x_ref[...], wr_ref[...], (((1,), (0,)), ((), ())),
        precision=lax.Precision.DEFAULT,
        preferred_element_type=jnp.float32) + br_ref[...]


def _tc_expert_body(x_ref, g_ref, wall_ref, be_ref, s_ref, t_ref, o_ref):
    y = lax.dot_general(
        x_ref[...], wall_ref[...], (((1,), (0,)), ((), ())),
        precision=lax.Precision.DEFAULT,
        preferred_element_type=jnp.float32) + be_ref[...]
    ws = lax.dot_general(
        g_ref[...], s_ref[...], (((1,), (0,)), ((), ())),
        precision=lax.Precision.HIGHEST,
        preferred_element_type=jnp.float32)
    o_ref[...] = lax.dot_general(
        y * ws, t_ref[...], (((1,), (0,)), ((), ())),
        precision=lax.Precision.HIGHEST,
        preferred_element_type=jnp.float32)


def _sc_route_body(n_tok, lg_hbm, g_hbm, lg_v, g_v):
    tok_w = n_tok // NW
    words = tok_w * NE
    blocks = tok_w // 16

    cid = lax.axis_index("c")
    sid = lax.axis_index("s")
    wid = sid * NC + cid

    pltpu.sync_copy(lg_hbm.at[pl.ds(wid * words, words)], lg_v)

    def vexp(v):
        t = v * jnp.float32(1.4426950408889634)
        t = jnp.maximum(t, jnp.float32(-126.0))
        magic = jnp.float32(12582912.0)
        n = (t + magic) - magic
        f = t - n
        ni = n.astype(jnp.int32)
        p = jnp.float32(0.00015403530393381608)
        p = p * f + jnp.float32(0.0013333558146428443)
        p = p * f + jnp.float32(0.009618129107628477)
        p = p * f + jnp.float32(0.05550410866482158)
        p = p * f + jnp.float32(0.2402265069591007)
        p = p * f + jnp.float32(0.6931471805599453)
        p = p * f + jnp.float32(1.0)
        scale = plsc.bitcast((ni + 127) << 23, jnp.float32)
        return p * scale

    iota = lax.iota(jnp.int32, 16)
    lane8 = iota * NE
    neg = jnp.full((16,), -3.0e38, jnp.float32)
    zero = jnp.zeros((16,), jnp.float32)

    def block(b, carry):
        base8 = lane8 + b * (16 * NE)
        logits = [plsc.load_gather(lg_v, [base8 + e]) for e in range(NE)]

        best1 = logits[0]
        idx1 = jnp.zeros((16,), jnp.int32)
        for e in range(1, NE):
            c = logits[e] > best1
            best1 = jnp.where(c, logits[e], best1)
            idx1 = jnp.where(c, jnp.int32(e), idx1)
        best2 = neg
        idx2 = jnp.zeros((16,), jnp.int32)
        for e in range(NE):
            le = jnp.where(idx1 == e, neg, logits[e])
            c = le > best2
            best2 = jnp.where(c, le, best2)
            idx2 = jnp.where(c, jnp.int32(e), idx2)

        s = zero
        for e in range(NE):
            s = s + vexp(logits[e] - best1)
        r = 1.0 / s
        r = r + r * (jnp.float32(1.0) - s * r)
        g1 = r
        g2 = vexp(best2 - best1) * r

        for e in range(NE):
            ge = jnp.where(idx1 == e, g1, jnp.where(idx2 == e, g2, zero))
            plsc.store_scatter(g_v, [base8 + e], ge)
        return carry

    lax.fori_loop(0, blocks, block, 0)
    pltpu.sync_copy(g_v, g_hbm.at[pl.ds(wid * words, words)])


@jax.jit
def _moe_call(x, Wr, br2, wall, be_row, sel, seg):
    n = x.shape[0]
    grid = n // BN

    logits = pl.pallas_call(
        _tc_logits_body,
        grid=(grid,),
        in_specs=[
            pl.BlockSpec((BN, EMB), lambda i: (i, 0)),
            pl.BlockSpec((EMB, NE), lambda i: (0, 0)),
            pl.BlockSpec((1, NE), lambda i: (0, 0)),
        ],
        out_specs=pl.BlockSpec((BN, NE), lambda i: (i, 0)),
        out_shape=jax.ShapeDtypeStruct((n, NE), jnp.float32),
    )(x, Wr, br2)

    words = (n // NW) * NE
    route = pl.kernel(
        functools.partial(_sc_route_body, n),
        out_type=jax.ShapeDtypeStruct((n * NE,), jnp.float32),
        mesh=plsc.VectorSubcoreMesh(core_axis_name="c", subcore_axis_name="s",
                                    num_cores=NC, num_subcores=NS),
        scratch_types=[
            pltpu.VMEM((words,), jnp.float32),
            pltpu.VMEM((words,), jnp.float32),
        ],
        compiler_params=pltpu.CompilerParams(needs_layout_passes=False),
    )
    gates = route(logits.reshape(-1)).reshape(n, NE)

    out = pl.pallas_call(
        _tc_expert_body,
        grid=(grid,),
        in_specs=[
            pl.BlockSpec((BN, EMB), lambda i: (i, 0)),
            pl.BlockSpec((BN, NE), lambda i: (i, 0)),
            pl.BlockSpec((EMB, NE * EMB), lambda i: (0, 0)),
            pl.BlockSpec((1, NE * EMB), lambda i: (0, 0)),
            pl.BlockSpec((NE, NE * EMB), lambda i: (0, 0)),
            pl.BlockSpec((NE * EMB, EMB), lambda i: (0, 0)),
        ],
        out_specs=pl.BlockSpec((BN, EMB), lambda i: (i, 0)),
        out_shape=jax.ShapeDtypeStruct((n, EMB), jnp.float32),
    )(x, gates, wall, be_row, sel, seg)
    return out


_SEL = np.repeat(np.eye(NE, dtype=np.float32), EMB, axis=1)
_SEG = np.tile(np.eye(EMB, dtype=np.float32), (NE, 1))


def kernel(x, Wr, br, We, be):
    wall = We.transpose(1, 0, 2).reshape(EMB, NE * EMB)
    be_row = be.reshape(1, NE * EMB)
    return _moe_call(x, Wr, br.reshape(1, NE), wall, be_row,
                     jnp.asarray(_SEL), jnp.asarray(_SEG))


# speedup vs baseline: 3.2254x; 1.3296x over previous
import functools

import jax
import jax.numpy as jnp
import numpy as np
from jax import lax
from jax.experimental import pallas as pl
from jax.experimental.pallas import tpu as pltpu
from jax.experimental.pallas import tpu_sc as plsc

EMB = 16
NE = 8
NC, NS = 2, 16
NW = NC * NS
BN = 4096


def _tc_logits_body(x_ref, wr_ref, br_ref, o_ref):
    o_ref[...] = lax.dot_general(
        x_ref[...], wr_ref[...], (((1,), (0,)), ((), ())),
        precision=lax.Precision.DEFAULT,
        preferred_element_type=jnp.float32) + br_ref[...]


def _tc_expert_body(x_ref, g_ref, wall_ref, be_ref, s_ref, t_ref, o_ref):
    y = lax.dot_general(
        x_ref[...], wall_ref[...], (((1,), (0,)), ((), ())),
        precision=lax.Precision.DEFAULT,
        preferred_element_type=jnp.float32) + be_ref[...]
    ws = lax.dot_general(
        g_ref[...], s_ref[...], (((1,), (0,)), ((), ())),
        precision=lax.Precision.DEFAULT,
        preferred_element_type=jnp.float32)
    o_ref[...] = lax.dot_general(
        y * ws, t_ref[...], (((1,), (0,)), ((), ())),
        precision=lax.Precision.DEFAULT,
        preferred_element_type=jnp.float32)


def _sc_route_body(n_tok, lg_hbm, g_hbm, lg_v, g_v):
    tok_w = n_tok // NW
    words = tok_w * NE
    blocks = tok_w // 16

    cid = lax.axis_index("c")
    sid = lax.axis_index("s")
    wid = sid * NC + cid

    pltpu.sync_copy(lg_hbm.at[pl.ds(wid * words, words)], lg_v)

    def vexp(v):
        t = v * jnp.float32(1.4426950408889634)
        t = jnp.maximum(t, jnp.float32(-126.0))
        magic = jnp.float32(12582912.0)
        n = (t + magic) - magic
        f = t - n
        ni = n.astype(jnp.int32)
        p = jnp.float32(0.00015403530393381608)
        p = p * f + jnp.float32(0.0013333558146428443)
        p = p * f + jnp.float32(0.009618129107628477)
        p = p * f + jnp.float32(0.05550410866482158)
        p = p * f + jnp.float32(0.2402265069591007)
        p = p * f + jnp.float32(0.6931471805599453)
        p = p * f + jnp.float32(1.0)
        scale = plsc.bitcast((ni + 127) << 23, jnp.float32)
        return p * scale

    iota = lax.iota(jnp.int32, 16)
    lane8 = iota * NE
    neg = jnp.full((16,), -3.0e38, jnp.float32)
    zero = jnp.zeros((16,), jnp.float32)

    def block(b, carry):
        base8 = lane8 + b * (16 * NE)
        logits = [plsc.load_gather(lg_v, [base8 + e]) for e in range(NE)]

        best1 = logits[0]
        idx1 = jnp.zeros((16,), jnp.int32)
        for e in range(1, NE):
            c = logits[e] > best1
            best1 = jnp.where(c, logits[e], best1)
            idx1 = jnp.where(c, jnp.int32(e), idx1)
        best2 = neg
        idx2 = jnp.zeros((16,), jnp.int32)
        for e in range(NE):
            le = jnp.where(idx1 == e, neg, logits[e])
            c = le > best2
            best2 = jnp.where(c, le, best2)
            idx2 = jnp.where(c, jnp.int32(e), idx2)

        s = zero
        for e in range(NE):
            s = s + vexp(logits[e] - best1)
        r = 1.0 / s
        r = r + r * (jnp.float32(1.0) - s * r)
        g1 = r
        g2 = vexp(best2 - best1) * r

        for e in range(NE):
            ge = jnp.where(idx1 == e, g1, jnp.where(idx2 == e, g2, zero))
            plsc.store_scatter(g_v, [base8 + e], ge)
        return carry

    lax.fori_loop(0, blocks, block, 0)
    pltpu.sync_copy(g_v, g_hbm.at[pl.ds(wid * words, words)])


@jax.jit
def _moe_call(x, Wr, br2, wall, be_row, sel, seg):
    n = x.shape[0]
    grid = n // BN

    logits = pl.pallas_call(
        _tc_logits_body,
        grid=(grid,),
        in_specs=[
            pl.BlockSpec((BN, EMB), lambda i: (i, 0)),
            pl.BlockSpec((EMB, NE), lambda i: (0, 0)),
            pl.BlockSpec((1, NE), lambda i: (0, 0)),
        ],
        out_specs=pl.BlockSpec((BN, NE), lambda i: (i, 0)),
        out_shape=jax.ShapeDtypeStruct((n, NE), jnp.float32),
    )(x, Wr, br2)

    words = (n // NW) * NE
    route = pl.kernel(
        functools.partial(_sc_route_body, n),
        out_type=jax.ShapeDtypeStruct((n * NE,), jnp.float32),
        mesh=plsc.VectorSubcoreMesh(core_axis_name="c", subcore_axis_name="s",
                                    num_cores=NC, num_subcores=NS),
        scratch_types=[
            pltpu.VMEM((words,), jnp.float32),
            pltpu.VMEM((words,), jnp.float32),
        ],
        compiler_params=pltpu.CompilerParams(needs_layout_passes=False),
    )
    gates = route(logits.reshape(-1)).reshape(n, NE)

    out = pl.pallas_call(
        _tc_expert_body,
        grid=(grid,),
        in_specs=[
            pl.BlockSpec((BN, EMB), lambda i: (i, 0)),
            pl.BlockSpec((BN, NE), lambda i: (i, 0)),
            pl.BlockSpec((EMB, NE * EMB), lambda i: (0, 0)),
            pl.BlockSpec((1, NE * EMB), lambda i: (0, 0)),
            pl.BlockSpec((NE, NE * EMB), lambda i: (0, 0)),
            pl.BlockSpec((NE * EMB, EMB), lambda i: (0, 0)),
        ],
        out_specs=pl.BlockSpec((BN, EMB), lambda i: (i, 0)),
        out_shape=jax.ShapeDtypeStruct((n, EMB), jnp.float32),
    )(x, gates, wall, be_row, sel, seg)
    return out


_SEL = np.repeat(np.eye(NE, dtype=np.float32), EMB, axis=1)
_SEG = np.tile(np.eye(EMB, dtype=np.float32), (NE, 1))


def kernel(x, Wr, br, We, be):
    wall = We.transpose(1, 0, 2).reshape(EMB, NE * EMB)
    be_row = be.reshape(1, NE * EMB)
    return _moe_call(x, Wr, br.reshape(1, NE), wall, be_row,
                     jnp.asarray(_SEL), jnp.asarray(_SEG))


# speedup vs baseline: 3.9086x; 1.2118x over previous
import functools

import jax
import jax.numpy as jnp
import numpy as np
from jax import lax
from jax.experimental import pallas as pl
from jax.experimental.pallas import tpu as pltpu
from jax.experimental.pallas import tpu_sc as plsc

EMB = 16
NE = 8
NC, NS = 2, 16
NW = NC * NS
BN = 4096


def _tc_logits_body(x_ref, wr_ref, br_ref, o_ref):
    o_ref[...] = lax.dot_general(
        x_ref[...], wr_ref[...], (((1,), (0,)), ((), ())),
        precision=lax.Precision.DEFAULT,
        preferred_element_type=jnp.float32) + br_ref[...]


def _tc_expert_body(x_ref, g_ref, wall_ref, be_ref, s_ref, t_ref, o_ref):
    y = lax.dot_general(
        x_ref[...], wall_ref[...], (((1,), (0,)), ((), ())),
        precision=lax.Precision.DEFAULT,
        preferred_element_type=jnp.float32) + be_ref[...]
    ws = lax.dot_general(
        g_ref[...], s_ref[...], (((1,), (0,)), ((), ())),
        precision=lax.Precision.DEFAULT,
        preferred_element_type=jnp.float32)
    o_ref[...] = lax.dot_general(
        y * ws, t_ref[...], (((1,), (0,)), ((), ())),
        precision=lax.Precision.DEFAULT,
        preferred_element_type=jnp.float32)


PACK = 16


def _sc_route_body(n_tok, lg_hbm, g_hbm, lg_v, g_v):
    tok_w = n_tok // NW
    words = tok_w * NE
    blocks = tok_w // 16

    cid = lax.axis_index("c")
    sid = lax.axis_index("s")
    wid = sid * NC + cid

    pltpu.sync_copy(lg_hbm.at[pl.ds(wid * words, words)], lg_v)

    def vexp(v):
        t = v * jnp.float32(1.4426950408889634)
        t = jnp.maximum(t, jnp.float32(-126.0))
        magic = jnp.float32(12582912.0)
        n = (t + magic) - magic
        f = t - n
        ni = n.astype(jnp.int32)
        p = jnp.float32(0.00015403530393381608)
        p = p * f + jnp.float32(0.0013333558146428443)
        p = p * f + jnp.float32(0.009618129107628477)
        p = p * f + jnp.float32(0.05550410866482158)
        p = p * f + jnp.float32(0.2402265069591007)
        p = p * f + jnp.float32(0.6931471805599453)
        p = p * f + jnp.float32(1.0)
        scale = plsc.bitcast((ni + 127) << 23, jnp.float32)
        return p * scale

    iota = lax.iota(jnp.int32, 16)
    lane8 = iota * NE
    neg = jnp.full((16,), -3.0e38, jnp.float32)
    zero = jnp.zeros((16,), jnp.float32)

    def block(b, carry):
        base8 = lane8 + b * (16 * NE)
        logits = [plsc.load_gather(lg_v, [base8 + e]) for e in range(NE)]

        best1 = logits[0]
        idx1 = jnp.zeros((16,), jnp.int32)
        for e in range(1, NE):
            c = logits[e] > best1
            best1 = jnp.where(c, logits[e], best1)
            idx1 = jnp.where(c, jnp.int32(e), idx1)
        best2 = neg
        idx2 = jnp.zeros((16,), jnp.int32)
        for e in range(NE):
            le = jnp.where(idx1 == e, neg, logits[e])
            c = le > best2
            best2 = jnp.where(c, le, best2)
            idx2 = jnp.where(c, jnp.int32(e), idx2)

        s = zero
        for e in range(NE):
            s = s + vexp(logits[e] - best1)
        r = 1.0 / s
        r = r + r * (jnp.float32(1.0) - s * r)
        g1 = r
        g2 = vexp(best2 - best1) * r

        for e in range(NE):
            ge = jnp.where(idx1 == e, g1, jnp.where(idx2 == e, g2, zero))
            plsc.store_scatter(g_v, [base8 + e], ge)
        return carry

    lax.fori_loop(0, blocks, block, 0)
    pltpu.sync_copy(g_v, g_hbm.at[pl.ds(wid * words, words)])


@jax.jit
def _moe_call(xp, wrbd, brt, wallbd, bet, sbd, tbd):
    rows = xp.shape[0]
    n = rows * PACK
    gl = PACK * NE

    logits = pl.pallas_call(
        _tc_logits_body,
        grid=(2,),
        in_specs=[
            pl.BlockSpec((rows // 2, PACK * EMB), lambda i: (i, 0)),
            pl.BlockSpec((PACK * EMB, gl), lambda i: (0, 0)),
            pl.BlockSpec((1, gl), lambda i: (0, 0)),
        ],
        out_specs=pl.BlockSpec((rows // 2, gl), lambda i: (i, 0)),
        out_shape=jax.ShapeDtypeStruct((rows, gl), jnp.float32),
    )(xp, wrbd, brt)

    words = (n // NW) * NE
    route = pl.kernel(
        functools.partial(_sc_route_body, n),
        out_type=jax.ShapeDtypeStruct((n * NE,), jnp.float32),
        mesh=plsc.VectorSubcoreMesh(core_axis_name="c", subcore_axis_name="s",
                                    num_cores=NC, num_subcores=NS),
        scratch_types=[
            pltpu.VMEM((words,), jnp.float32),
            pltpu.VMEM((words,), jnp.float32),
        ],
        compiler_params=pltpu.CompilerParams(needs_layout_passes=False),
    )
    gates = route(logits.reshape(-1)).reshape(rows, gl)

    out = pl.pallas_call(
        _tc_expert_body,
        grid=(4,),
        in_specs=[
            pl.BlockSpec((rows // 4, PACK * EMB), lambda i: (i, 0)),
            pl.BlockSpec((rows // 4, gl), lambda i: (i, 0)),
            pl.BlockSpec((PACK * EMB, PACK * NE * EMB), lambda i: (0, 0)),
            pl.BlockSpec((1, PACK * NE * EMB), lambda i: (0, 0)),
            pl.BlockSpec((gl, PACK * NE * EMB), lambda i: (0, 0)),
            pl.BlockSpec((PACK * NE * EMB, PACK * EMB), lambda i: (0, 0)),
        ],
        out_specs=pl.BlockSpec((rows // 4, PACK * EMB), lambda i: (i, 0)),
        out_shape=jax.ShapeDtypeStruct((rows, PACK * EMB), jnp.float32),
    )(xp, gates, wallbd, bet, sbd, tbd)
    return out


_EYEP = np.eye(PACK, dtype=np.float32)
_SEL = np.repeat(np.eye(NE, dtype=np.float32), EMB, axis=1)
_SEG = np.tile(np.eye(EMB, dtype=np.float32), (NE, 1))
_SBD = np.kron(_EYEP, _SEL)
_TBD = np.kron(_EYEP, _SEG)


def kernel(x, Wr, br, We, be):
    n, emb = x.shape
    xp = x.reshape(n // PACK, PACK * emb)
    eyep = jnp.asarray(_EYEP)
    wrbd = jnp.kron(eyep, Wr)
    brt = jnp.tile(br, PACK).reshape(1, PACK * NE)
    wall = We.transpose(1, 0, 2).reshape(EMB, NE * EMB)
    wallbd = jnp.kron(eyep, wall)
    bet = jnp.tile(be.reshape(-1), PACK).reshape(1, PACK * NE * EMB)
    out = _moe_call(xp, wrbd, brt, wallbd, bet,
                    jnp.asarray(_SBD), jnp.asarray(_TBD))
    return out.reshape(n, emb)
